# baseline (speedup 1.0000x reference)
import functools

import jax
import jax.numpy as jnp
from jax import lax
from jax.experimental import pallas as pl
from jax.experimental.pallas import tpu as pltpu
from jax.experimental.pallas import tpu_sc as plsc

EMBED_DIM = 64


@functools.lru_cache(maxsize=None)
def _make_gather(nb: int, ns_pad: int):
    D = EMBED_DIM
    info = plsc.get_sparse_core_info()
    NC, NS, L = info.num_cores, info.num_subcores, info.num_lanes
    NW = NC * NS
    assert nb % NW == 0
    b_per_w = nb // NW
    GB = 4
    CH = GB * ns_pad
    n_ch = b_per_w // GB
    B2 = nb * ns_pad

    mesh = plsc.VectorSubcoreMesh(core_axis_name="c", subcore_axis_name="s")

    @functools.partial(
        pl.kernel,
        mesh=mesh,
        out_type=jax.ShapeDtypeStruct((B2, 2 * D), jnp.float32),
        scratch_types=[
            pltpu.VMEM((b_per_w * ns_pad,), jnp.int32),
            pltpu.VMEM((CH, 2 * D), jnp.float32),
            pltpu.SemaphoreType.DMA,
        ],
        compiler_params=pltpu.CompilerParams(needs_layout_passes=False),
    )
    def gather_kernel(t128_hbm, idx_hbm, out_hbm, idx_v, rows_v, sem):
        wid = lax.axis_index("s") * NC + lax.axis_index("c")
        base = wid * b_per_w * ns_pad
        pltpu.sync_copy(idx_hbm.at[pl.ds(base, b_per_w * ns_pad)], idx_v)

        def chunk(i, _):
            pltpu.async_copy(
                t128_hbm.at[idx_v.at[pl.ds(i * CH, CH)]], rows_v, sem
            ).wait()
            pltpu.sync_copy(rows_v, out_hbm.at[pl.ds(base + i * CH, CH)])
            return _
        lax.fori_loop(0, n_ch, chunk, None)

    return gather_kernel


def kernel(table, inputs):
    nb, ns = inputs.shape
    D = table.shape[1]
    ns_pad = (ns + 7) // 8 * 8
    idx = jnp.pad(inputs.astype(jnp.int32), ((0, 0), (0, ns_pad - ns))).reshape(-1)
    table128 = jnp.pad(table, ((0, 0), (0, D)))
    out2 = _make_gather(nb, ns_pad)(table128, idx)
    return out2.reshape(nb, ns_pad, 2 * D)[:, :ns, :D]


# speedup vs baseline: 2.4768x; 2.4768x over previous
import functools

import jax
import jax.numpy as jnp
from jax import lax
from jax.experimental import pallas as pl
from jax.experimental.pallas import tpu as pltpu
from jax.experimental.pallas import tpu_sc as plsc

EMBED_DIM = 64


@functools.lru_cache(maxsize=None)
def _make_gather(nb: int, ns_pad: int):
    D = EMBED_DIM
    info = plsc.get_sparse_core_info()
    NC, NS, L = info.num_cores, info.num_subcores, info.num_lanes
    NW = NC * NS
    assert nb % NW == 0
    b_per_w = nb // NW
    GB = 4
    CH = GB * ns_pad
    n_ch = b_per_w // GB
    B2 = nb * ns_pad

    mesh = plsc.VectorSubcoreMesh(core_axis_name="c", subcore_axis_name="s")

    @functools.partial(
        pl.kernel,
        mesh=mesh,
        out_type=jax.ShapeDtypeStruct((B2, 2 * D), jnp.float32),
        scratch_types=[
            pltpu.VMEM((b_per_w * ns_pad,), jnp.int32),
            pltpu.VMEM((CH, 2 * D), jnp.float32),
            pltpu.SemaphoreType.DMA,
        ],
        compiler_params=pltpu.CompilerParams(needs_layout_passes=False),
    )
    def gather_kernel(t128_hbm, idx_hbm, out_hbm, idx_v, rows_v, sem):
        wid = lax.axis_index("s") * NC + lax.axis_index("c")
        base = wid * b_per_w * ns_pad
        pltpu.sync_copy(idx_hbm.at[pl.ds(base, b_per_w * ns_pad)], idx_v)

        def chunk(i, _):
            pltpu.async_copy(
                t128_hbm.at[idx_v.at[pl.ds(i * CH, CH)]], rows_v, sem
            ).wait()
            pltpu.sync_copy(rows_v, out_hbm.at[pl.ds(base + i * CH, CH)])
            return _
        lax.fori_loop(0, n_ch, chunk, None)

    return gather_kernel


def kernel(table, inputs):
    nb, ns = inputs.shape
    D = table.shape[1]
    ns_pad = (ns + 7) // 8 * 8
    idx = jnp.pad(
        inputs.astype(jnp.int32), ((0, 0), (0, ns_pad - ns)), mode="edge"
    ).reshape(-1)
    table128 = jnp.pad(table, ((0, 0), (0, D)))
    out2 = _make_gather(nb, ns_pad)(table128, idx)
    return out2.reshape(nb, ns_pad, 2 * D)[:, :ns, :D]


# speedup vs baseline: 2.6010x; 1.0501x over previous
import functools

import jax
import jax.numpy as jnp
from jax import lax
from jax.experimental import pallas as pl
from jax.experimental.pallas import tpu as pltpu
from jax.experimental.pallas import tpu_sc as plsc

EMBED_DIM = 64


@functools.lru_cache(maxsize=None)
def _make_gather(nb: int, ns_pad: int):
    D = EMBED_DIM
    info = plsc.get_sparse_core_info()
    NC, NS, L = info.num_cores, info.num_subcores, info.num_lanes
    NW = NC * NS
    assert nb % NW == 0
    b_per_w = nb // NW
    GB = 8
    CH = GB * ns_pad
    n_ch = b_per_w // GB
    assert n_ch % 2 == 0
    B2 = nb * ns_pad

    mesh = plsc.VectorSubcoreMesh(core_axis_name="c", subcore_axis_name="s")

    @functools.partial(
        pl.kernel,
        mesh=mesh,
        out_type=jax.ShapeDtypeStruct((B2, 2 * D), jnp.float32),
        scratch_types=[
            pltpu.VMEM((b_per_w * ns_pad,), jnp.int32),
            [pltpu.VMEM((CH, 2 * D), jnp.float32)] * 2,
            [pltpu.SemaphoreType.DMA] * 2,
            [pltpu.SemaphoreType.DMA] * 2,
        ],
        compiler_params=pltpu.CompilerParams(needs_layout_passes=False),
    )
    def gather_kernel(t128_hbm, idx_hbm, out_hbm, idx_v, rows_v, gsem, ssem):
        wid = lax.axis_index("s") * NC + lax.axis_index("c")
        base = wid * b_per_w * ns_pad
        pltpu.sync_copy(idx_hbm.at[pl.ds(base, b_per_w * ns_pad)], idx_v)

        def gather_start(i, b):
            return pltpu.async_copy(
                t128_hbm.at[idx_v.at[pl.ds(i * CH, CH)]], rows_v[b], gsem[b]
            )

        def scatter_start(i, b):
            return pltpu.async_copy(
                rows_v[b], out_hbm.at[pl.ds(base + i * CH, CH)], ssem[b]
            )

        gather_start(0, 0)

        for i in range(n_ch):
            b = i % 2
            if i + 1 < n_ch:
                if i >= 1:
                    pltpu.make_async_copy(
                        rows_v[1 - b],
                        out_hbm.at[pl.ds(base + (i - 1) * CH, CH)],
                        ssem[1 - b],
                    ).wait()
                gather_start(i + 1, 1 - b)
            pltpu.make_async_copy(
                t128_hbm.at[idx_v.at[pl.ds(i * CH, CH)]], rows_v[b], gsem[b]
            ).wait()
            scatter_start(i, b)
        pltpu.make_async_copy(
            rows_v[(n_ch - 2) % 2],
            out_hbm.at[pl.ds(base + (n_ch - 2) * CH, CH)],
            ssem[(n_ch - 2) % 2],
        ).wait()
        pltpu.make_async_copy(
            rows_v[(n_ch - 1) % 2],
            out_hbm.at[pl.ds(base + (n_ch - 1) * CH, CH)],
            ssem[(n_ch - 1) % 2],
        ).wait()

    return gather_kernel


def kernel(table, inputs):
    nb, ns = inputs.shape
    D = table.shape[1]
    ns_pad = (ns + 7) // 8 * 8
    idx = jnp.pad(
        inputs.astype(jnp.int32), ((0, 0), (0, ns_pad - ns)), mode="edge"
    ).reshape(-1)
    table128 = jnp.pad(table, ((0, 0), (0, D)))
    out2 = _make_gather(nb, ns_pad)(table128, idx)
    return out2.reshape(nb, ns_pad, 2 * D)[:, :ns, :D]
